# baseline (speedup 1.0000x reference)
import functools

import jax
import jax.numpy as jnp
from jax import lax
from jax.experimental import pallas as pl
from jax.experimental.pallas import tpu as pltpu
from jax.experimental.pallas import tpu_sc as plsc

_L = 16


def kernel(patch, pos_table):
    B, P, D = patch.shape
    R = B * D
    xt = jnp.transpose(patch, (0, 2, 1))
    x2 = xt.reshape(R, P)
    pos2 = jnp.transpose(pos_table, (1, 0))

    NRG = 16
    NCH = 2
    WR = R // NRG
    WC = P // NCH
    CR = 8
    NCHUNK = WR // CR
    NBUF = 4
    mesh = plsc.VectorSubcoreMesh(core_axis_name="c", subcore_axis_name="s")

    @functools.partial(
        pl.kernel,
        mesh=mesh,
        out_type=jax.ShapeDtypeStruct((R, P), jnp.float32),
        scratch_types=[
            pltpu.VMEM((D, WC), jnp.float32),
            pltpu.VMEM((CR, WC), jnp.float32),
            pltpu.VMEM((CR, WC), jnp.float32),
            pltpu.VMEM((CR, WC), jnp.float32),
            pltpu.VMEM((CR, WC), jnp.float32),
            pltpu.SemaphoreType.DMA,
            pltpu.SemaphoreType.DMA,
            pltpu.SemaphoreType.DMA,
            pltpu.SemaphoreType.DMA,
            pltpu.SemaphoreType.DMA,
            pltpu.SemaphoreType.DMA,
            pltpu.SemaphoreType.DMA,
            pltpu.SemaphoreType.DMA,
        ],
    )
    def sc_add(x_hbm, pos_hbm, out_hbm, pos_v, b0, b1, b2, b3,
               si0, si1, si2, si3, so0, so1, so2, so3):
        wid = lax.axis_index("s") * 2 + lax.axis_index("c")
        rg = wid // NCH
        ch = wid % NCH
        r0 = rg * WR
        c0 = ch * WC
        cslc = pl.ds(c0, WC)
        bufs = (b0, b1, b2, b3)
        sins = (si0, si1, si2, si3)
        souts = (so0, so1, so2, so3)
        pltpu.sync_copy(pos_hbm.at[:, cslc], pos_v)
        pltpu.async_copy(x_hbm.at[pl.ds(r0, CR), cslc], b0, si0)
        pltpu.async_copy(x_hbm.at[pl.ds(r0 + CR, CR), cslc], b1, si1)

        def body(i, carry):
            for ph in range(NBUF):
                k = i * NBUF + ph
                row = r0 + k * CR
                pr = lax.rem(k * CR, D)
                pltpu.make_async_copy(
                    x_hbm.at[pl.ds(row, CR), cslc], bufs[ph], sins[ph]
                ).wait()

                nxt = (ph + 2) % NBUF

                @pl.when(k >= 2)
                def _free_next_buf():
                    pltpu.make_async_copy(
                        bufs[nxt], out_hbm.at[pl.ds(row, CR), cslc], souts[nxt]
                    ).wait()

                @pl.when(k + 2 < NCHUNK)
                def _start_next_in():
                    pltpu.async_copy(
                        x_hbm.at[pl.ds(row + 2 * CR, CR), cslc],
                        bufs[nxt], sins[nxt],
                    )

                for r in range(CR):
                    for j in range(WC // _L):
                        sl = pl.ds(j * _L, _L)
                        bufs[ph][r, sl] = bufs[ph][r, sl] + pos_v[pr + r, sl]
                pltpu.async_copy(
                    bufs[ph], out_hbm.at[pl.ds(row, CR), cslc], souts[ph]
                )
            return carry

        lax.fori_loop(0, NCHUNK // NBUF, body, 0)
        r_last = r0 + (NCHUNK - 2) * CR
        pltpu.make_async_copy(
            bufs[2], out_hbm.at[pl.ds(r_last, CR), cslc], souts[2]
        ).wait()
        pltpu.make_async_copy(
            bufs[3], out_hbm.at[pl.ds(r_last + CR, CR), cslc], souts[3]
        ).wait()

    out2 = sc_add(x2, pos2)
    return jnp.transpose(out2.reshape(B, D, P), (0, 2, 1))


# speedup vs baseline: 6.2051x; 6.2051x over previous
import jax
import jax.numpy as jnp
from jax.experimental import pallas as pl


def _add_body(x_ref, pos_ref, o_ref):
    o_ref[...] = x_ref[...] + pos_ref[...][None]


def kernel(patch, pos_table):
    B, P, D = patch.shape
    xt = jnp.transpose(patch, (0, 2, 1))
    post = jnp.transpose(pos_table, (1, 0))
    BB = 32
    DD = 48
    out_t = pl.pallas_call(
        _add_body,
        grid=(B // BB, D // DD),
        in_specs=[
            pl.BlockSpec((BB, DD, P), lambda i, j: (i, j, 0)),
            pl.BlockSpec((DD, P), lambda i, j: (j, 0)),
        ],
        out_specs=pl.BlockSpec((BB, DD, P), lambda i, j: (i, j, 0)),
        out_shape=jax.ShapeDtypeStruct((B, D, P), jnp.float32),
    )(xt, post)
    return jnp.transpose(out_t, (0, 2, 1))


# speedup vs baseline: 6.8679x; 1.1068x over previous
import jax
import jax.numpy as jnp
from jax.experimental import pallas as pl


def _add_body(x_ref, pos_ref, o_ref):
    o_ref[...] = x_ref[...] + pos_ref[...][None]


def kernel(patch, pos_table):
    B, P, D = patch.shape
    xt = jnp.transpose(patch, (0, 2, 1))
    post = jnp.transpose(pos_table, (1, 0))
    BB = 32
    out_t = pl.pallas_call(
        _add_body,
        grid=(B // BB,),
        in_specs=[
            pl.BlockSpec((BB, D, P), lambda i: (i, 0, 0)),
            pl.BlockSpec((D, P), lambda i: (0, 0)),
        ],
        out_specs=pl.BlockSpec((BB, D, P), lambda i: (i, 0, 0)),
        out_shape=jax.ShapeDtypeStruct((B, D, P), jnp.float32),
    )(xt, post)
    return jnp.transpose(out_t, (0, 2, 1))
